# baseline (speedup 1.0000x reference)
import jax
import jax.numpy as jnp
from jax.experimental import pallas as pl
from jax.experimental.pallas import tpu as pltpu

N = 4096
D = 256
H = 256
ALPHA = 0.1
RATE = 0.5

BM = 256
NH = N // 2
GRID = N // BM

_BF = jnp.bfloat16
_F32 = jnp.float32


def _lrelu(x):
    return jnp.where(x > 0, x, ALPHA * x)


def _body(vus_lo, vus_hi, vut_lo, vut_hi, uvs_lo, uvs_hi, uvt_lo, uvt_hi,
          xs_ref, xt_ref,
          w1_ref, b1_ref, w2_ref, b2_ref, w3_ref, b3_ref, w4_ref, b4_ref,
          wsua_ref, wsub_ref, bsu_ref, wtua_ref, wtub_ref, btu_ref,
          out_ref, s1s_scr, s1t_scr, s2s_scr, s2t_scr):
    i = pl.program_id(0)

    @pl.when(i == 0)
    def _():
        s1s_scr[...] = jnp.dot(xs_ref[...], w1_ref[...].astype(_BF),
                               preferred_element_type=_F32)
        s1t_scr[...] = jnp.dot(xt_ref[...], w2_ref[...].astype(_BF),
                               preferred_element_type=_F32)

    @pl.when(i < GRID)
    def _():
        row = i * BM
        h1s = _lrelu(jnp.dot(vus_lo[...], s1s_scr[:NH, :],
                             preferred_element_type=_F32)
                     + jnp.dot(vus_hi[...], s1s_scr[NH:, :],
                               preferred_element_type=_F32)
                     + b1_ref[...])
        s2s_scr[pl.ds(row, BM), :] = jnp.dot(
            h1s.astype(_BF), w3_ref[...].astype(_BF),
            preferred_element_type=_F32)
        h1t = _lrelu(jnp.dot(vut_lo[...], s1t_scr[:NH, :],
                             preferred_element_type=_F32)
                     + jnp.dot(vut_hi[...], s1t_scr[NH:, :],
                               preferred_element_type=_F32)
                     + b2_ref[...])
        s2t_scr[pl.ds(row, BM), :] = jnp.dot(
            h1t.astype(_BF), w4_ref[...].astype(_BF),
            preferred_element_type=_F32)

    @pl.when(i >= GRID)
    def _():
        row = (i - GRID) * BM
        o2s = _lrelu(jnp.dot(uvs_lo[...], s2s_scr[:NH, :],
                             preferred_element_type=_F32)
                     + jnp.dot(uvs_hi[...], s2s_scr[NH:, :],
                               preferred_element_type=_F32)
                     + b3_ref[...])
        o2t = _lrelu(jnp.dot(uvt_lo[...], s2t_scr[:NH, :],
                             preferred_element_type=_F32)
                     + jnp.dot(uvt_hi[...], s2t_scr[NH:, :],
                               preferred_element_type=_F32)
                     + b4_ref[...])
        lin_s = (jnp.dot(o2s.astype(_BF), wsua_ref[...], preferred_element_type=_F32)
                 + jnp.dot(xs_ref[pl.ds(row, BM), :], wsub_ref[...],
                           preferred_element_type=_F32)
                 + bsu_ref[...])
        lin_t = (jnp.dot(o2t.astype(_BF), wtua_ref[...], preferred_element_type=_F32)
                 + jnp.dot(xt_ref[pl.ds(row, BM), :], wtub_ref[...],
                           preferred_element_type=_F32)
                 + btu_ref[...])
        out_ref[...] = RATE * jax.nn.relu(lin_s) + (1.0 - RATE) * jax.nn.relu(lin_t)


def kernel(source_ufea, target_ufea, source_UV_adj, source_VU_adj, target_UV_adj,
           target_VU_adj, W1, b1, W2, b2, W3, b3, W4, b4, Wsu, bsu, Wtu, btu):
    xs_bf = source_ufea.astype(_BF)
    xt_bf = target_ufea.astype(_BF)
    b1r = b1.reshape(1, H)
    b2r = b2.reshape(1, H)
    b3r = b3.reshape(1, D)
    b4r = b4.reshape(1, D)
    bsur = bsu.reshape(1, D)
    btur = btu.reshape(1, D)
    wsua = Wsu[:, :H].T.astype(_BF)
    wsub = Wsu[:, H:].T.astype(_BF)
    wtua = Wtu[:, :H].T.astype(_BF)
    wtub = Wtu[:, H:].T.astype(_BF)

    full = lambda shape: pl.BlockSpec(shape, lambda i: (0, 0))
    vu_lo = pl.BlockSpec((BM, NH), lambda i: (jnp.minimum(i, GRID - 1), 0))
    vu_hi = pl.BlockSpec((BM, NH), lambda i: (jnp.minimum(i, GRID - 1), 1))
    uv_lo = pl.BlockSpec((BM, NH), lambda i: (jnp.maximum(i - GRID, 0), 0))
    uv_hi = pl.BlockSpec((BM, NH), lambda i: (jnp.maximum(i - GRID, 0), 1))
    out_spec = pl.BlockSpec((BM, D), lambda i: (jnp.maximum(i - GRID, 0), 0))

    out = pl.pallas_call(
        _body,
        grid=(2 * GRID,),
        in_specs=[
            vu_lo, vu_hi, vu_lo, vu_hi,
            uv_lo, uv_hi, uv_lo, uv_hi,
            full((N, D)), full((N, D)),
            full((D, H)), full((1, H)),
            full((D, H)), full((1, H)),
            full((H, D)), full((1, D)),
            full((H, D)), full((1, D)),
            full((H, D)), full((D, D)), full((1, D)),
            full((H, D)), full((D, D)), full((1, D)),
        ],
        out_specs=out_spec,
        out_shape=jax.ShapeDtypeStruct((N, D), _F32),
        scratch_shapes=[pltpu.VMEM((N, H), _F32), pltpu.VMEM((N, H), _F32),
                        pltpu.VMEM((N, D), _F32), pltpu.VMEM((N, D), _F32)],
        compiler_params=pltpu.CompilerParams(
            dimension_semantics=("arbitrary",)),
    )(source_VU_adj, source_VU_adj, target_VU_adj, target_VU_adj,
      source_UV_adj, source_UV_adj, target_UV_adj, target_UV_adj,
      xs_bf, xt_bf, W1, b1r, W2, b2r, W3, b3r, W4, b4r,
      wsua, wsub, bsur, wtua, wtub, btur)

    return (out, out)


# speedup vs baseline: 1.0045x; 1.0045x over previous
import jax
import jax.numpy as jnp
from jax.experimental import pallas as pl
from jax.experimental.pallas import tpu as pltpu

N = 4096
D = 256
H = 256
ALPHA = 0.1
RATE = 0.5

BM = 256
GRID = N // BM

_BF = jnp.bfloat16
_F32 = jnp.float32


def _lrelu(x):
    return jnp.where(x > 0, x, ALPHA * x)


def _body(vus_ref, vut_ref, uvs_ref, uvt_ref, xs_ref, xt_ref,
          w1_ref, b1_ref, w2_ref, b2_ref, w3_ref, b3_ref, w4_ref, b4_ref,
          wsua_ref, wsub_ref, bsu_ref, wtua_ref, wtub_ref, btu_ref,
          out_ref, s1s_scr, s1t_scr, s2s_scr, s2t_scr):
    i = pl.program_id(0)

    @pl.when(i == 0)
    def _():
        s1s_scr[...] = jnp.dot(xs_ref[...], w1_ref[...].astype(_BF),
                               preferred_element_type=_F32).astype(_BF)
        s1t_scr[...] = jnp.dot(xt_ref[...], w2_ref[...].astype(_BF),
                               preferred_element_type=_F32).astype(_BF)

    @pl.when(i < GRID)
    def _():
        row = i * BM
        h1s = _lrelu(jnp.dot(vus_ref[...], s1s_scr[...],
                             preferred_element_type=_F32) + b1_ref[...])
        s2s_scr[pl.ds(row, BM), :] = jnp.dot(
            h1s.astype(_BF), w3_ref[...].astype(_BF),
            preferred_element_type=_F32).astype(_BF)
        h1t = _lrelu(jnp.dot(vut_ref[...], s1t_scr[...],
                             preferred_element_type=_F32) + b2_ref[...])
        s2t_scr[pl.ds(row, BM), :] = jnp.dot(
            h1t.astype(_BF), w4_ref[...].astype(_BF),
            preferred_element_type=_F32).astype(_BF)

    @pl.when(i >= GRID)
    def _():
        row = (i - GRID) * BM
        o2s = _lrelu(jnp.dot(uvs_ref[...], s2s_scr[...],
                             preferred_element_type=_F32) + b3_ref[...])
        o2t = _lrelu(jnp.dot(uvt_ref[...], s2t_scr[...],
                             preferred_element_type=_F32) + b4_ref[...])
        lin_s = (jnp.dot(o2s.astype(_BF), wsua_ref[...], preferred_element_type=_F32)
                 + jnp.dot(xs_ref[pl.ds(row, BM), :], wsub_ref[...],
                           preferred_element_type=_F32)
                 + bsu_ref[...])
        lin_t = (jnp.dot(o2t.astype(_BF), wtua_ref[...], preferred_element_type=_F32)
                 + jnp.dot(xt_ref[pl.ds(row, BM), :], wtub_ref[...],
                           preferred_element_type=_F32)
                 + btu_ref[...])
        out_ref[...] = RATE * jax.nn.relu(lin_s) + (1.0 - RATE) * jax.nn.relu(lin_t)


def kernel(source_ufea, target_ufea, source_UV_adj, source_VU_adj, target_UV_adj,
           target_VU_adj, W1, b1, W2, b2, W3, b3, W4, b4, Wsu, bsu, Wtu, btu):
    b1r = b1.reshape(1, H)
    b2r = b2.reshape(1, H)
    b3r = b3.reshape(1, D)
    b4r = b4.reshape(1, D)
    bsur = bsu.reshape(1, D)
    btur = btu.reshape(1, D)
    wsua = Wsu[:, :H].T.astype(_BF)
    wsub = Wsu[:, H:].T.astype(_BF)
    wtua = Wtu[:, :H].T.astype(_BF)
    wtub = Wtu[:, H:].T.astype(_BF)

    full = lambda shape: pl.BlockSpec(shape, lambda i: (0, 0))
    vu_spec = pl.BlockSpec((BM, N), lambda i: (jnp.minimum(i, GRID - 1), 0))
    uv_spec = pl.BlockSpec((BM, N), lambda i: (jnp.maximum(i - GRID, 0), 0))
    out_spec = pl.BlockSpec((BM, D), lambda i: (jnp.maximum(i - GRID, 0), 0))

    out = pl.pallas_call(
        _body,
        grid=(2 * GRID,),
        in_specs=[
            vu_spec, vu_spec,
            uv_spec, uv_spec,
            full((N, D)), full((N, D)),
            full((D, H)), full((1, H)),
            full((D, H)), full((1, H)),
            full((H, D)), full((1, D)),
            full((H, D)), full((1, D)),
            full((H, D)), full((D, D)), full((1, D)),
            full((H, D)), full((D, D)), full((1, D)),
        ],
        out_specs=out_spec,
        out_shape=jax.ShapeDtypeStruct((N, D), _F32),
        scratch_shapes=[pltpu.VMEM((N, H), _BF), pltpu.VMEM((N, H), _BF),
                        pltpu.VMEM((N, D), _BF), pltpu.VMEM((N, D), _BF)],
        compiler_params=pltpu.CompilerParams(
            dimension_semantics=("arbitrary",)),
    )(source_VU_adj, target_VU_adj, source_UV_adj, target_UV_adj,
      source_ufea.astype(_BF), target_ufea.astype(_BF), W1, b1r, W2, b2r, W3, b3r, W4, b4r,
      wsua, wsub, bsur, wtua, wtub, btur)

    return (out, out)


# speedup vs baseline: 1.0265x; 1.0219x over previous
import jax
import jax.numpy as jnp
from jax.experimental import pallas as pl
from jax.experimental.pallas import tpu as pltpu

N = 4096
D = 256
H = 256
ALPHA = 0.1
RATE = 0.5

BM = 256
GRID = N // BM

_BF = jnp.bfloat16
_F32 = jnp.float32


def _lrelu(x):
    return jnp.where(x > 0, x, ALPHA * x)


def _stage1_body(vus_ref, vut_ref, xs_ref, xt_ref, w1_ref, b1_ref, w2_ref, b2_ref,
                 h1s_ref, h1t_ref, s1s_scr, s1t_scr):
    @pl.when(pl.program_id(0) == 0)
    def _():
        s1s_scr[...] = jnp.dot(xs_ref[...], w1_ref[...].astype(_BF),
                               preferred_element_type=_F32).astype(_BF)
        s1t_scr[...] = jnp.dot(xt_ref[...], w2_ref[...].astype(_BF),
                               preferred_element_type=_F32).astype(_BF)

    acc_s = jnp.dot(vus_ref[...], s1s_scr[...],
                    preferred_element_type=_F32) + b1_ref[...]
    h1s_ref[...] = _lrelu(acc_s).astype(_BF)
    acc_t = jnp.dot(vut_ref[...], s1t_scr[...],
                    preferred_element_type=_F32) + b2_ref[...]
    h1t_ref[...] = _lrelu(acc_t).astype(_BF)


def _stage2_body(uvs_ref, uvt_ref, h1s_ref, h1t_ref, xs_ref, xt_ref,
                 w3_ref, b3_ref, w4_ref, b4_ref,
                 wsua_ref, wsub_ref, bsu_ref, wtua_ref, wtub_ref, btu_ref,
                 out_ref, s2s_scr, s2t_scr):
    i = pl.program_id(0)

    @pl.when(i == 0)
    def _():
        s2s_scr[...] = jnp.dot(h1s_ref[...], w3_ref[...].astype(_BF),
                               preferred_element_type=_F32).astype(_BF)
        s2t_scr[...] = jnp.dot(h1t_ref[...], w4_ref[...].astype(_BF),
                               preferred_element_type=_F32).astype(_BF)

    o2s = _lrelu(jnp.dot(uvs_ref[...], s2s_scr[...],
                         preferred_element_type=_F32) + b3_ref[...])
    o2t = _lrelu(jnp.dot(uvt_ref[...], s2t_scr[...],
                         preferred_element_type=_F32) + b4_ref[...])

    lin_s = (jnp.dot(o2s.astype(_BF), wsua_ref[...], preferred_element_type=_F32)
             + jnp.dot(xs_ref[...], wsub_ref[...], preferred_element_type=_F32)
             + bsu_ref[...])
    lin_t = (jnp.dot(o2t.astype(_BF), wtua_ref[...], preferred_element_type=_F32)
             + jnp.dot(xt_ref[...], wtub_ref[...], preferred_element_type=_F32)
             + btu_ref[...])
    out_ref[...] = RATE * jax.nn.relu(lin_s) + (1.0 - RATE) * jax.nn.relu(lin_t)


def kernel(source_ufea, target_ufea, source_UV_adj, source_VU_adj, target_UV_adj,
           target_VU_adj, W1, b1, W2, b2, W3, b3, W4, b4, Wsu, bsu, Wtu, btu):
    xs_bf = source_ufea.astype(_BF)
    xt_bf = target_ufea.astype(_BF)
    b1r = b1.reshape(1, H)
    b2r = b2.reshape(1, H)
    b3r = b3.reshape(1, D)
    b4r = b4.reshape(1, D)
    bsur = bsu.reshape(1, D)
    btur = btu.reshape(1, D)
    wsua = Wsu[:, :H].T.astype(_BF)
    wsub = Wsu[:, H:].T.astype(_BF)
    wtua = Wtu[:, :H].T.astype(_BF)
    wtub = Wtu[:, H:].T.astype(_BF)

    full = lambda shape: pl.BlockSpec(shape, lambda i: (0, 0))
    rows = lambda shape: pl.BlockSpec(shape, lambda i: (i, 0))

    h1s, h1t = pl.pallas_call(
        _stage1_body,
        grid=(GRID,),
        in_specs=[
            rows((BM, N)), rows((BM, N)),
            full((N, D)), full((N, D)),
            full((D, H)), full((1, H)),
            full((D, H)), full((1, H)),
        ],
        out_specs=[rows((BM, H)), rows((BM, H))],
        out_shape=[jax.ShapeDtypeStruct((N, H), _BF),
                   jax.ShapeDtypeStruct((N, H), _BF)],
        scratch_shapes=[pltpu.VMEM((N, H), _BF), pltpu.VMEM((N, H), _BF)],
        compiler_params=pltpu.CompilerParams(
            dimension_semantics=("arbitrary",)),
    )(source_VU_adj, target_VU_adj, xs_bf, xt_bf, W1, b1r, W2, b2r)

    out = pl.pallas_call(
        _stage2_body,
        grid=(GRID,),
        in_specs=[
            rows((BM, N)), rows((BM, N)),
            full((N, H)), full((N, H)),
            rows((BM, D)), rows((BM, D)),
            full((H, D)), full((1, D)),
            full((H, D)), full((1, D)),
            full((H, D)), full((D, D)), full((1, D)),
            full((H, D)), full((D, D)), full((1, D)),
        ],
        out_specs=rows((BM, D)),
        out_shape=jax.ShapeDtypeStruct((N, D), _F32),
        scratch_shapes=[pltpu.VMEM((N, D), _BF), pltpu.VMEM((N, D), _BF)],
        compiler_params=pltpu.CompilerParams(
            dimension_semantics=("arbitrary",)),
    )(source_UV_adj, target_UV_adj, h1s, h1t, xs_bf, xt_bf,
      W3, b3r, W4, b4r, wsua, wsub, bsur, wtua, wtub, btur)

    return (out, out)


# speedup vs baseline: 1.0269x; 1.0004x over previous
import jax
import jax.numpy as jnp
from jax.experimental import pallas as pl
from jax.experimental.pallas import tpu as pltpu

N = 4096
D = 256
H = 256
ALPHA = 0.1
RATE = 0.5

BM = 256
GRID = N // BM

_BF = jnp.bfloat16
_F32 = jnp.float32


def _lrelu(x):
    return jnp.where(x > 0, x, ALPHA * x)


def _stage1_body(vus_ref, vut_ref, xs_ref, xt_ref, w1_ref, b1_ref, w2_ref, b2_ref,
                 h1s_ref, h1t_ref, s1s_scr, s1t_scr):
    @pl.when(pl.program_id(0) == 0)
    def _():
        s1s_scr[...] = jnp.dot(xs_ref[...], w1_ref[...].astype(_BF),
                               preferred_element_type=_F32).astype(_BF)
        s1t_scr[...] = jnp.dot(xt_ref[...], w2_ref[...].astype(_BF),
                               preferred_element_type=_F32).astype(_BF)

    acc_s = jnp.dot(vus_ref[...].astype(_BF), s1s_scr[...],
                    preferred_element_type=_F32) + b1_ref[...]
    h1s_ref[...] = _lrelu(acc_s).astype(_BF)
    acc_t = jnp.dot(vut_ref[...].astype(_BF), s1t_scr[...],
                    preferred_element_type=_F32) + b2_ref[...]
    h1t_ref[...] = _lrelu(acc_t).astype(_BF)


def _stage2_body(uvs_ref, uvt_ref, h1s_ref, h1t_ref, xs_ref, xt_ref,
                 w3_ref, b3_ref, w4_ref, b4_ref,
                 wsua_ref, wsub_ref, bsu_ref, wtua_ref, wtub_ref, btu_ref,
                 out_ref, s2s_scr, s2t_scr):
    i = pl.program_id(0)

    @pl.when(i == 0)
    def _():
        s2s_scr[...] = jnp.dot(h1s_ref[...], w3_ref[...].astype(_BF),
                               preferred_element_type=_F32).astype(_BF)
        s2t_scr[...] = jnp.dot(h1t_ref[...], w4_ref[...].astype(_BF),
                               preferred_element_type=_F32).astype(_BF)

    o2s = _lrelu(jnp.dot(uvs_ref[...].astype(_BF), s2s_scr[...],
                         preferred_element_type=_F32) + b3_ref[...])
    o2t = _lrelu(jnp.dot(uvt_ref[...].astype(_BF), s2t_scr[...],
                         preferred_element_type=_F32) + b4_ref[...])

    lin_s = (jnp.dot(o2s.astype(_BF), wsua_ref[...], preferred_element_type=_F32)
             + jnp.dot(xs_ref[...], wsub_ref[...], preferred_element_type=_F32)
             + bsu_ref[...])
    lin_t = (jnp.dot(o2t.astype(_BF), wtua_ref[...], preferred_element_type=_F32)
             + jnp.dot(xt_ref[...], wtub_ref[...], preferred_element_type=_F32)
             + btu_ref[...])
    out_ref[...] = RATE * jax.nn.relu(lin_s) + (1.0 - RATE) * jax.nn.relu(lin_t)


def kernel(source_ufea, target_ufea, source_UV_adj, source_VU_adj, target_UV_adj,
           target_VU_adj, W1, b1, W2, b2, W3, b3, W4, b4, Wsu, bsu, Wtu, btu):
    xs_bf = source_ufea.astype(_BF)
    xt_bf = target_ufea.astype(_BF)
    b1r = b1.reshape(1, H)
    b2r = b2.reshape(1, H)
    b3r = b3.reshape(1, D)
    b4r = b4.reshape(1, D)
    bsur = bsu.reshape(1, D)
    btur = btu.reshape(1, D)
    wsua = Wsu[:, :H].T.astype(_BF)
    wsub = Wsu[:, H:].T.astype(_BF)
    wtua = Wtu[:, :H].T.astype(_BF)
    wtub = Wtu[:, H:].T.astype(_BF)

    full = lambda shape: pl.BlockSpec(shape, lambda i: (0, 0))
    rows = lambda shape: pl.BlockSpec(shape, lambda i: (i, 0))

    h1s, h1t = pl.pallas_call(
        _stage1_body,
        grid=(GRID,),
        in_specs=[
            rows((BM, N)), rows((BM, N)),
            full((N, D)), full((N, D)),
            full((D, H)), full((1, H)),
            full((D, H)), full((1, H)),
        ],
        out_specs=[rows((BM, H)), rows((BM, H))],
        out_shape=[jax.ShapeDtypeStruct((N, H), _BF),
                   jax.ShapeDtypeStruct((N, H), _BF)],
        scratch_shapes=[pltpu.VMEM((N, H), _BF), pltpu.VMEM((N, H), _BF)],
        compiler_params=pltpu.CompilerParams(
            dimension_semantics=("arbitrary",)),
    )(source_VU_adj, target_VU_adj, xs_bf, xt_bf, W1, b1r, W2, b2r)

    out = pl.pallas_call(
        _stage2_body,
        grid=(GRID,),
        in_specs=[
            rows((BM, N)), rows((BM, N)),
            full((N, H)), full((N, H)),
            rows((BM, D)), rows((BM, D)),
            full((H, D)), full((1, D)),
            full((H, D)), full((1, D)),
            full((H, D)), full((D, D)), full((1, D)),
            full((H, D)), full((D, D)), full((1, D)),
        ],
        out_specs=rows((BM, D)),
        out_shape=jax.ShapeDtypeStruct((N, D), _F32),
        scratch_shapes=[pltpu.VMEM((N, D), _BF), pltpu.VMEM((N, D), _BF)],
        compiler_params=pltpu.CompilerParams(
            dimension_semantics=("arbitrary",)),
    )(source_UV_adj, target_UV_adj, h1s, h1t, xs_bf, xt_bf,
      W3, b3r, W4, b4r, wsua, wsub, bsur, wtua, wtub, btur)

    return (out, out)


# speedup vs baseline: 1.0593x; 1.0315x over previous
import jax
import jax.numpy as jnp
from jax.experimental import pallas as pl
from jax.experimental.pallas import tpu as pltpu

N = 4096
D = 256
H = 256
ALPHA = 0.1
RATE = 0.5

BM = 256
GRID = N // BM

_BF = jnp.bfloat16
_F32 = jnp.float32


def _lrelu(x):
    return jnp.where(x > 0, x, ALPHA * x)


def _stage1_body(vus_ref, vut_ref, xs_ref, xt_ref, w1_ref, b1_ref, w2_ref, b2_ref,
                 h1s_ref, h1t_ref, s1s_scr, s1t_scr):
    @pl.when(pl.program_id(0) == 0)
    def _():
        s1s_scr[...] = jnp.dot(xs_ref[...].astype(_BF), w1_ref[...].astype(_BF),
                               preferred_element_type=_F32).astype(_BF)
        s1t_scr[...] = jnp.dot(xt_ref[...].astype(_BF), w2_ref[...].astype(_BF),
                               preferred_element_type=_F32).astype(_BF)

    acc_s = jnp.dot(vus_ref[...].astype(_BF), s1s_scr[...],
                    preferred_element_type=_F32) + b1_ref[...]
    h1s_ref[...] = _lrelu(acc_s).astype(_BF)
    acc_t = jnp.dot(vut_ref[...].astype(_BF), s1t_scr[...],
                    preferred_element_type=_F32) + b2_ref[...]
    h1t_ref[...] = _lrelu(acc_t).astype(_BF)


def _stage2_body(uvs_ref, uvt_ref, h1s_ref, h1t_ref, xs_ref, xt_ref,
                 w3_ref, b3_ref, w4_ref, b4_ref,
                 wsua_ref, wsub_ref, bsu_ref, wtua_ref, wtub_ref, btu_ref,
                 out_ref, s2s_scr, s2t_scr):
    i = pl.program_id(0)

    @pl.when(i == 0)
    def _():
        s2s_scr[...] = jnp.dot(h1s_ref[...], w3_ref[...].astype(_BF),
                               preferred_element_type=_F32).astype(_BF)
        s2t_scr[...] = jnp.dot(h1t_ref[...], w4_ref[...].astype(_BF),
                               preferred_element_type=_F32).astype(_BF)

    o2s = _lrelu(jnp.dot(uvs_ref[...].astype(_BF), s2s_scr[...],
                         preferred_element_type=_F32) + b3_ref[...])
    o2t = _lrelu(jnp.dot(uvt_ref[...].astype(_BF), s2t_scr[...],
                         preferred_element_type=_F32) + b4_ref[...])

    lin_s = (jnp.dot(o2s.astype(_BF), wsua_ref[...], preferred_element_type=_F32)
             + jnp.dot(xs_ref[...].astype(_BF), wsub_ref[...], preferred_element_type=_F32)
             + bsu_ref[...])
    lin_t = (jnp.dot(o2t.astype(_BF), wtua_ref[...], preferred_element_type=_F32)
             + jnp.dot(xt_ref[...].astype(_BF), wtub_ref[...], preferred_element_type=_F32)
             + btu_ref[...])
    out_ref[...] = RATE * jax.nn.relu(lin_s) + (1.0 - RATE) * jax.nn.relu(lin_t)


def kernel(source_ufea, target_ufea, source_UV_adj, source_VU_adj, target_UV_adj,
           target_VU_adj, W1, b1, W2, b2, W3, b3, W4, b4, Wsu, bsu, Wtu, btu):
    b1r = b1.reshape(1, H)
    b2r = b2.reshape(1, H)
    b3r = b3.reshape(1, D)
    b4r = b4.reshape(1, D)
    bsur = bsu.reshape(1, D)
    btur = btu.reshape(1, D)
    wsua = Wsu[:, :H].T.astype(_BF)
    wsub = Wsu[:, H:].T.astype(_BF)
    wtua = Wtu[:, :H].T.astype(_BF)
    wtub = Wtu[:, H:].T.astype(_BF)

    full = lambda shape: pl.BlockSpec(shape, lambda i: (0, 0))
    rows = lambda shape: pl.BlockSpec(shape, lambda i: (i, 0))

    h1s, h1t = pl.pallas_call(
        _stage1_body,
        grid=(GRID,),
        in_specs=[
            rows((BM, N)), rows((BM, N)),
            full((N, D)), full((N, D)),
            full((D, H)), full((1, H)),
            full((D, H)), full((1, H)),
        ],
        out_specs=[rows((BM, H)), rows((BM, H))],
        out_shape=[jax.ShapeDtypeStruct((N, H), _BF),
                   jax.ShapeDtypeStruct((N, H), _BF)],
        scratch_shapes=[pltpu.VMEM((N, H), _BF), pltpu.VMEM((N, H), _BF)],
        compiler_params=pltpu.CompilerParams(
            dimension_semantics=("arbitrary",)),
    )(source_VU_adj, target_VU_adj, source_ufea, target_ufea, W1, b1r, W2, b2r)

    out = pl.pallas_call(
        _stage2_body,
        grid=(GRID,),
        in_specs=[
            rows((BM, N)), rows((BM, N)),
            full((N, H)), full((N, H)),
            rows((BM, D)), rows((BM, D)),
            full((H, D)), full((1, D)),
            full((H, D)), full((1, D)),
            full((H, D)), full((D, D)), full((1, D)),
            full((H, D)), full((D, D)), full((1, D)),
        ],
        out_specs=rows((BM, D)),
        out_shape=jax.ShapeDtypeStruct((N, D), _F32),
        scratch_shapes=[pltpu.VMEM((N, D), _BF), pltpu.VMEM((N, D), _BF)],
        compiler_params=pltpu.CompilerParams(
            dimension_semantics=("arbitrary",)),
    )(source_UV_adj, target_UV_adj, h1s, h1t, source_ufea, target_ufea,
      W3, b3r, W4, b4r, wsua, wsub, bsur, wtua, wtub, btur)

    return (out, out)
